# baseline (speedup 1.0000x reference)
import jax
import jax.numpy as jnp
import numpy as np
from jax import lax
from jax.experimental import pallas as pl
from jax.experimental.pallas import tpu as pltpu
from jax.experimental.pallas import tpu_sc as plsc

_K = 128
_N = 32768
_ROWS = 128
_NW = 32
_ROWS_PER_W = _ROWS // _NW
_NV = _N // 16
_INT_MIN = np.int32(-2147483648)


def _sc_body(x_hbm, out_hbm, row_ref, cand_ref, histc_ref, res_ref):
    wid = lax.axis_index("s") * 2 + lax.axis_index("c")
    lanes = lax.iota(jnp.int32, 16)
    ones = jnp.ones((16,), jnp.int32)
    zeros_i = jnp.zeros((16,), jnp.int32)
    zeros_f = jnp.zeros((16,), jnp.float32)

    res = zeros_f
    for j in range(_ROWS_PER_W):
        row = wid * _ROWS_PER_W + j
        pltpu.sync_copy(x_hbm.at[row], row_ref)

        def zero_body(i, _):
            histc_ref[pl.ds(i * 16, 16)] = zeros_i
            return 0
        lax.fori_loop(0, 256, zero_body, 0)

        def hist_body(i, _):
            xv = row_ref[pl.ds(i * 16, 16)]
            b = lax.bitcast_convert_type(xv, jnp.int32)
            k = jnp.where(b < 0, _INT_MIN - b, b)
            digit = lax.shift_right_logical(k ^ _INT_MIN, 24)
            plsc.addupdate_scatter(histc_ref, [(digit << 4) + lanes], ones)
            return 0
        lax.fori_loop(0, _NV, hist_body, 0)

        def phb_body(i, carry):
            c_cum, d_star, c_above = carry
            d = 255 - i
            cnt = jnp.sum(histc_ref[pl.ds(d * 16, 16)])
            new_cum = c_cum + cnt
            found = (c_cum < _K) & (new_cum >= _K)
            d_star = jnp.where(found, d, d_star)
            c_above = jnp.where(found, c_cum, c_above)
            return (new_cum, d_star, c_above)
        _, d_star, c_above = lax.fori_loop(
            0, 256, phb_body, (jnp.int32(0), jnp.int32(0), jnp.int32(0)))

        d_star_v = jnp.full((16,), d_star, jnp.int32)

        def p2_body(i, carry):
            ptr, acc = carry
            xv = row_ref[pl.ds(i * 16, 16)]
            b = lax.bitcast_convert_type(xv, jnp.int32)
            k = jnp.where(b < 0, _INT_MIN - b, b)
            digit = lax.shift_right_logical(k ^ _INT_MIN, 24)
            acc = acc + jnp.where(digit > d_star_v, xv, 0.0)
            m_eq = digit == d_star_v
            plsc.store_compressed(cand_ref.at[pl.ds(ptr, 16)], k, mask=m_eq)
            ptr = ptr + jnp.sum(m_eq.astype(jnp.int32))
            return (ptr, acc)
        ptr, acc = lax.fori_loop(0, _NV, p2_body, (jnp.int32(0), zeros_f))
        s_above = jnp.sum(acc)

        cand_ref[pl.ds(ptr, 16)] = jnp.full((16,), _INT_MIN, jnp.int32)
        nv = (ptr + 15) // 16
        r = _K - c_above

        def bis_body(i, t_u):
            cand_u = t_u | lax.shift_left(jnp.int32(1), 31 - i)
            spl = jnp.full((16,), cand_u ^ _INT_MIN, jnp.int32)

            def cnt_body(v, a):
                ck = cand_ref[pl.ds(v * 16, 16)]
                return a + (ck >= spl).astype(jnp.int32)
            a = lax.fori_loop(0, nv, cnt_body, zeros_i)
            return jnp.where(jnp.sum(a) >= r, cand_u, t_u)
        t_u = lax.fori_loop(0, 32, bis_body, jnp.int32(0))
        t_s = t_u ^ _INT_MIN
        t_v = jnp.full((16,), t_s, jnp.int32)

        def fin_body(v, carry):
            sc, cc = carry
            ck = cand_ref[pl.ds(v * 16, 16)]
            m = ck > t_v
            xv = lax.bitcast_convert_type(jnp.where(ck < 0, _INT_MIN - ck, ck), jnp.float32)
            return (sc + jnp.where(m, xv, 0.0), cc + m.astype(jnp.int32))
        sc, cc = lax.fori_loop(0, nv, fin_body, (zeros_f, zeros_i))

        t_val = lax.bitcast_convert_type(jnp.where(t_v < 0, _INT_MIN - t_v, t_v),
                             jnp.float32)
        total = (jnp.full((16,), s_above + jnp.sum(sc), jnp.float32)
                 + jnp.full((16,), (r - jnp.sum(cc)).astype(jnp.float32),
                            jnp.float32) * t_val)
        res = jnp.where(lanes == j, total * np.float32(1.0 / _K), res)

    res_ref[...] = res
    pltpu.sync_copy(res_ref, out_hbm.at[wid])


import functools


@functools.cache
def _sc_call():
    return pl.kernel(
        _sc_body,
        out_type=jax.ShapeDtypeStruct((_NW, 16), jnp.float32),
        mesh=plsc.VectorSubcoreMesh(core_axis_name="c", subcore_axis_name="s",
                                    num_cores=2, num_subcores=16),
        compiler_params=pltpu.CompilerParams(needs_layout_passes=False),
        scratch_types=[
            pltpu.VMEM((_N,), jnp.float32),
            pltpu.VMEM((_N + 16,), jnp.int32),
            pltpu.VMEM((4096,), jnp.int32),
            pltpu.VMEM((16,), jnp.float32),
        ],
    )


def kernel(x):
    out = _sc_call()(x)
    return out[:, :_ROWS_PER_W].reshape(_ROWS)


# speedup vs baseline: 1.0584x; 1.0584x over previous
import jax
import jax.numpy as jnp
import numpy as np
from jax import lax
from jax.experimental import pallas as pl
from jax.experimental.pallas import tpu as pltpu
from jax.experimental.pallas import tpu_sc as plsc

_K = 128
_N = 32768
_ROWS = 128
_NW = 32
_ROWS_PER_W = _ROWS // _NW
_NV = _N // 16
_INT_MIN = np.int32(-2147483648)


def _sc_body(x_hbm, out_hbm, row_ref, cand_ref, histc_ref, res_ref):
    wid = lax.axis_index("s") * 2 + lax.axis_index("c")
    lanes = lax.iota(jnp.int32, 16)
    ones = jnp.ones((16,), jnp.int32)
    zeros_i = jnp.zeros((16,), jnp.int32)
    zeros_f = jnp.zeros((16,), jnp.float32)

    res = zeros_f
    for j in range(_ROWS_PER_W):
        row = wid * _ROWS_PER_W + j
        pltpu.sync_copy(x_hbm.at[row], row_ref)

        def zero_body(i, _):
            histc_ref[pl.ds(i * 16, 16)] = zeros_i
            return 0
        lax.fori_loop(0, 256, zero_body, 0)

        def hist_body(i, _):
            base = i * 128
            for u in range(8):
                xv = row_ref[pl.ds(base + u * 16, 16)]
                b = lax.bitcast_convert_type(xv, jnp.int32)
                k = jnp.where(b < 0, _INT_MIN - b, b)
                digit = lax.shift_right_logical(k ^ _INT_MIN, 24)
                plsc.addupdate_scatter(histc_ref, [(digit << 4) + lanes], ones)
            return 0
        lax.fori_loop(0, _NV // 8, hist_body, 0)

        def phb_body(i, carry):
            for u in range(4):
                c_cum, d_star, c_above = carry
                d = 255 - (i * 4 + u)
                cnt = jnp.sum(histc_ref[pl.ds(d * 16, 16)])
                new_cum = c_cum + cnt
                found = (c_cum < _K) & (new_cum >= _K)
                d_star = jnp.where(found, d, d_star)
                c_above = jnp.where(found, c_cum, c_above)
                carry = (new_cum, d_star, c_above)
            return carry
        _, d_star, c_above = lax.fori_loop(
            0, 64, phb_body, (jnp.int32(0), jnp.int32(0), jnp.int32(0)))

        d_star_v = jnp.full((16,), d_star, jnp.int32)

        def p2_body(i, carry):
            cnt_v, acc = carry
            base = i * 128
            for u in range(8):
                xv = row_ref[pl.ds(base + u * 16, 16)]
                b = lax.bitcast_convert_type(xv, jnp.int32)
                k = jnp.where(b < 0, _INT_MIN - b, b)
                digit = lax.shift_right_logical(k ^ _INT_MIN, 24)
                acc = acc + jnp.where(digit > d_star_v, xv, 0.0)
                m_eq = digit == d_star_v
                plsc.store_scatter(cand_ref, [(cnt_v << 4) + lanes], k,
                                   mask=m_eq)
                cnt_v = cnt_v + m_eq.astype(jnp.int32)
            return (cnt_v, acc)
        cnt_v, acc = lax.fori_loop(0, _NV // 8, p2_body, (zeros_i, zeros_f))
        s_above = jnp.sum(acc)

        min_c = jnp.min(cnt_v)
        nv = jnp.max(cnt_v)
        sentinel = jnp.full((16,), _INT_MIN, jnp.int32)

        def fill_body(v, _):
            old = cand_ref[pl.ds(v * 16, 16)]
            keep = jnp.full((16,), v, jnp.int32) < cnt_v
            cand_ref[pl.ds(v * 16, 16)] = jnp.where(keep, old, sentinel)
            return 0
        lax.fori_loop(min_c, nv, fill_body, 0)
        r = _K - c_above

        def bis_body(i, t_u):
            cand_u = t_u | lax.shift_left(jnp.int32(1), 31 - i)
            spl = jnp.full((16,), cand_u ^ _INT_MIN, jnp.int32)

            def cnt_body(v, a):
                ck = cand_ref[pl.ds(v * 16, 16)]
                return a + (ck >= spl).astype(jnp.int32)
            a = lax.fori_loop(0, nv, cnt_body, zeros_i)
            return jnp.where(jnp.sum(a) >= r, cand_u, t_u)
        t_u = lax.fori_loop(0, 32, bis_body, jnp.int32(0))
        t_s = t_u ^ _INT_MIN
        t_v = jnp.full((16,), t_s, jnp.int32)

        def fin_body(v, carry):
            sc, cc = carry
            ck = cand_ref[pl.ds(v * 16, 16)]
            m = ck > t_v
            xv = lax.bitcast_convert_type(jnp.where(ck < 0, _INT_MIN - ck, ck), jnp.float32)
            return (sc + jnp.where(m, xv, 0.0), cc + m.astype(jnp.int32))
        sc, cc = lax.fori_loop(0, nv, fin_body, (zeros_f, zeros_i))

        t_val = lax.bitcast_convert_type(jnp.where(t_v < 0, _INT_MIN - t_v, t_v),
                             jnp.float32)
        total = (jnp.full((16,), s_above + jnp.sum(sc), jnp.float32)
                 + jnp.full((16,), (r - jnp.sum(cc)).astype(jnp.float32),
                            jnp.float32) * t_val)
        res = jnp.where(lanes == j, total * np.float32(1.0 / _K), res)

    res_ref[...] = res
    pltpu.sync_copy(res_ref, out_hbm.at[wid])


import functools


@functools.cache
def _sc_call():
    return pl.kernel(
        _sc_body,
        out_type=jax.ShapeDtypeStruct((_NW, 16), jnp.float32),
        mesh=plsc.VectorSubcoreMesh(core_axis_name="c", subcore_axis_name="s",
                                    num_cores=2, num_subcores=16),
        compiler_params=pltpu.CompilerParams(needs_layout_passes=False),
        scratch_types=[
            pltpu.VMEM((_N,), jnp.float32),
            pltpu.VMEM((_N + 16,), jnp.int32),
            pltpu.VMEM((4096,), jnp.int32),
            pltpu.VMEM((16,), jnp.float32),
        ],
    )


def kernel(x):
    out = _sc_call()(x)
    return out[:, :_ROWS_PER_W].reshape(_ROWS)


# speedup vs baseline: 2.2223x; 2.0996x over previous
import jax
import jax.numpy as jnp
import numpy as np
from jax import lax
from jax.experimental import pallas as pl
from jax.experimental.pallas import tpu as pltpu
from jax.experimental.pallas import tpu_sc as plsc

_K = 128
_N = 32768
_ROWS = 128
_NW = 32
_ROWS_PER_W = _ROWS // _NW
_NV = _N // 16
_INT_MIN = np.int32(-2147483648)


def _sc_body(x_hbm, out_hbm, row_ref, cand_ref, histc_ref, res_ref):
    wid = lax.axis_index("s") * 2 + lax.axis_index("c")
    lanes = lax.iota(jnp.int32, 16)
    ones = jnp.ones((16,), jnp.int32)
    zeros_i = jnp.zeros((16,), jnp.int32)
    zeros_f = jnp.zeros((16,), jnp.float32)

    res = zeros_f
    for j in range(_ROWS_PER_W):
        row = wid * _ROWS_PER_W + j
        pltpu.sync_copy(x_hbm.at[row], row_ref)

        def zero_body(i, _):
            histc_ref[pl.ds(i * 16, 16)] = zeros_i
            return 0
        lax.fori_loop(0, 256, zero_body, 0)

        @plsc.parallel_loop(0, _NV, 1, unroll=8)
        def _(i):
            xv = row_ref[pl.ds(i * 16, 16)]
            b = lax.bitcast_convert_type(xv, jnp.int32)
            k = jnp.where(b < 0, _INT_MIN - b, b)
            digit = lax.shift_right_logical(k ^ _INT_MIN, 24)
            plsc.addupdate_scatter(histc_ref, [(digit << 4) + lanes], ones)

        def phb_body(i, carry):
            for u in range(4):
                c_cum, d_star, c_above = carry
                d = 255 - (i * 4 + u)
                cnt = jnp.sum(histc_ref[pl.ds(d * 16, 16)])
                new_cum = c_cum + cnt
                found = (c_cum < _K) & (new_cum >= _K)
                d_star = jnp.where(found, d, d_star)
                c_above = jnp.where(found, c_cum, c_above)
                carry = (new_cum, d_star, c_above)
            return carry
        _, d_star, c_above = lax.fori_loop(
            0, 64, phb_body, (jnp.int32(0), jnp.int32(0), jnp.int32(0)))

        d_star_v = jnp.full((16,), d_star, jnp.int32)

        @plsc.parallel_loop(0, _NV, 1, unroll=8, carry=(zeros_i, zeros_f))
        def p2_out(i, carry):
            cnt_v, acc = carry
            xv = row_ref[pl.ds(i * 16, 16)]
            b = lax.bitcast_convert_type(xv, jnp.int32)
            k = jnp.where(b < 0, _INT_MIN - b, b)
            digit = lax.shift_right_logical(k ^ _INT_MIN, 24)
            acc = acc + jnp.where(digit > d_star_v, xv, 0.0)
            m_eq = digit == d_star_v
            plsc.store_scatter(cand_ref, [(cnt_v << 4) + lanes], k,
                               mask=m_eq)
            return (cnt_v + m_eq.astype(jnp.int32), acc)
        cnt_v, acc = p2_out
        s_above = jnp.sum(acc)

        min_c = jnp.min(cnt_v)
        nv = jnp.max(cnt_v)
        sentinel = jnp.full((16,), _INT_MIN, jnp.int32)

        def fill_body(v, _):
            old = cand_ref[pl.ds(v * 16, 16)]
            keep = jnp.full((16,), v, jnp.int32) < cnt_v
            cand_ref[pl.ds(v * 16, 16)] = jnp.where(keep, old, sentinel)
            return 0
        lax.fori_loop(min_c, nv, fill_body, 0)
        r = _K - c_above

        def bis_body(i, t_u):
            cand_u = t_u | lax.shift_left(jnp.int32(1), 31 - i)
            spl = jnp.full((16,), cand_u ^ _INT_MIN, jnp.int32)

            def cnt_body(v, a):
                ck = cand_ref[pl.ds(v * 16, 16)]
                return a + (ck >= spl).astype(jnp.int32)
            a = lax.fori_loop(0, nv, cnt_body, zeros_i)
            return jnp.where(jnp.sum(a) >= r, cand_u, t_u)
        t_u = lax.fori_loop(0, 32, bis_body, jnp.int32(0))
        t_s = t_u ^ _INT_MIN
        t_v = jnp.full((16,), t_s, jnp.int32)

        def fin_body(v, carry):
            sc, cc = carry
            ck = cand_ref[pl.ds(v * 16, 16)]
            m = ck > t_v
            xv = lax.bitcast_convert_type(jnp.where(ck < 0, _INT_MIN - ck, ck), jnp.float32)
            return (sc + jnp.where(m, xv, 0.0), cc + m.astype(jnp.int32))
        sc, cc = lax.fori_loop(0, nv, fin_body, (zeros_f, zeros_i))

        t_val = lax.bitcast_convert_type(jnp.where(t_v < 0, _INT_MIN - t_v, t_v),
                             jnp.float32)
        total = (jnp.full((16,), s_above + jnp.sum(sc), jnp.float32)
                 + jnp.full((16,), (r - jnp.sum(cc)).astype(jnp.float32),
                            jnp.float32) * t_val)
        res = jnp.where(lanes == j, total * np.float32(1.0 / _K), res)

    res_ref[...] = res
    pltpu.sync_copy(res_ref, out_hbm.at[wid])


import functools


@functools.cache
def _sc_call():
    return pl.kernel(
        _sc_body,
        out_type=jax.ShapeDtypeStruct((_NW, 16), jnp.float32),
        mesh=plsc.VectorSubcoreMesh(core_axis_name="c", subcore_axis_name="s",
                                    num_cores=2, num_subcores=16),
        compiler_params=pltpu.CompilerParams(needs_layout_passes=False),
        scratch_types=[
            pltpu.VMEM((_N,), jnp.float32),
            pltpu.VMEM((_N + 16,), jnp.int32),
            pltpu.VMEM((4096,), jnp.int32),
            pltpu.VMEM((16,), jnp.float32),
        ],
    )


def kernel(x):
    out = _sc_call()(x)
    return out[:, :_ROWS_PER_W].reshape(_ROWS)


# speedup vs baseline: 2.9371x; 1.3217x over previous
import functools

import jax
import jax.numpy as jnp
import numpy as np
from jax import lax
from jax.experimental import pallas as pl
from jax.experimental.pallas import tpu as pltpu
from jax.experimental.pallas import tpu_sc as plsc

_K = 128
_N = 32768
_ROWS = 128
_NW = 32
_ROWS_PER_W = _ROWS // _NW
_NV = _N // 16
_INT_MIN = np.int32(-2147483648)


def _sc_body(x_hbm, out_hbm, rowa_ref, rowb_ref, cand_ref, histc_ref,
             res_ref, sema, semb):
    wid = lax.axis_index("s") * 2 + lax.axis_index("c")
    lanes = lax.iota(jnp.int32, 16)
    ones = jnp.ones((16,), jnp.int32)
    zeros_i = jnp.zeros((16,), jnp.int32)
    zeros_f = jnp.zeros((16,), jnp.float32)
    bufs = (rowa_ref, rowb_ref)
    sems = (sema, semb)
    base_row = wid * _ROWS_PER_W

    @plsc.parallel_loop(0, 256, 1, unroll=8)
    def _(i):
        histc_ref[pl.ds(i * 16, 16)] = zeros_i

    cp = pltpu.async_copy(x_hbm.at[base_row], bufs[0], sems[0])
    res = zeros_f
    for j in range(_ROWS_PER_W):
        row_ref = bufs[j % 2]
        if j + 1 < _ROWS_PER_W:
            nxt = pltpu.async_copy(x_hbm.at[base_row + j + 1],
                                   bufs[(j + 1) % 2], sems[(j + 1) % 2])
        cp.wait()

        @plsc.parallel_loop(0, _NV, 1, unroll=8)
        def _(i):
            xv = row_ref[pl.ds(i * 16, 16)]
            b = lax.bitcast_convert_type(xv, jnp.int32)
            k = jnp.where(b < 0, _INT_MIN - b, b)
            digit = lax.shift_right_logical(k ^ _INT_MIN, 24)
            plsc.addupdate_scatter(histc_ref, [(digit << 4) + lanes], ones)

        def phb_body(i, carry):
            for u in range(4):
                c_cum, d_star = carry
                d = 255 - (i * 4 + u)
                cnt = jnp.sum(histc_ref[pl.ds(d * 16, 16)])
                histc_ref[pl.ds(d * 16, 16)] = zeros_i
                new_cum = c_cum + cnt
                found = (c_cum < _K) & (new_cum >= _K)
                d_star = jnp.where(found, d, d_star)
                carry = (new_cum, d_star)
            return carry
        _, d_star = lax.fori_loop(0, 64, phb_body,
                                  (jnp.int32(0), jnp.int32(0)))

        lo_v = jnp.full((16,), (d_star << 24) ^ _INT_MIN, jnp.int32)

        @plsc.parallel_loop(0, _NV, 1, unroll=8, carry=zeros_i)
        def cnt_v(i, cnt):
            xv = row_ref[pl.ds(i * 16, 16)]
            b = lax.bitcast_convert_type(xv, jnp.int32)
            k = jnp.where(b < 0, _INT_MIN - b, b)
            m = k >= lo_v
            plsc.store_scatter(cand_ref, [(cnt << 4) + lanes], k, mask=m)
            return cnt + m.astype(jnp.int32)

        min_c = jnp.min(cnt_v)
        nv = jnp.max(cnt_v)
        sentinel = jnp.full((16,), _INT_MIN, jnp.int32)

        def fill_body(v, _):
            old = cand_ref[pl.ds(v * 16, 16)]
            keep = jnp.full((16,), v, jnp.int32) < cnt_v
            cand_ref[pl.ds(v * 16, 16)] = jnp.where(keep, old, sentinel)
            return 0
        lax.fori_loop(min_c, nv, fill_body, 0)

        def bis_body(i, t_u):
            cand_u = t_u | lax.shift_left(jnp.int32(1), 23 - i)
            spl = jnp.full((16,), cand_u ^ _INT_MIN, jnp.int32)

            def cnt_body(v, a):
                ck = cand_ref[pl.ds(v * 16, 16)]
                return a + (ck >= spl).astype(jnp.int32)
            a = lax.fori_loop(0, nv, cnt_body, zeros_i)
            return jnp.where(jnp.sum(a) >= _K, cand_u, t_u)
        t_u = lax.fori_loop(0, 24, bis_body, d_star << 24)
        t_v = jnp.full((16,), t_u ^ _INT_MIN, jnp.int32)

        def fin_body(v, carry):
            sc, cc = carry
            ck = cand_ref[pl.ds(v * 16, 16)]
            m = ck > t_v
            xv = lax.bitcast_convert_type(
                jnp.where(ck < 0, _INT_MIN - ck, ck), jnp.float32)
            return (sc + jnp.where(m, xv, 0.0), cc + m.astype(jnp.int32))
        sc, cc = lax.fori_loop(0, nv, fin_body, (zeros_f, zeros_i))

        t_val = lax.bitcast_convert_type(
            jnp.where(t_v < 0, _INT_MIN - t_v, t_v), jnp.float32)
        total = (jnp.full((16,), jnp.sum(sc), jnp.float32)
                 + jnp.full((16,), (_K - jnp.sum(cc)).astype(jnp.float32),
                            jnp.float32) * t_val)
        res = jnp.where(lanes == j, total * np.float32(1.0 / _K), res)
        if j + 1 < _ROWS_PER_W:
            cp = nxt

    res_ref[...] = res
    pltpu.sync_copy(res_ref, out_hbm.at[wid])


@functools.cache
def _sc_call():
    return pl.kernel(
        _sc_body,
        out_type=jax.ShapeDtypeStruct((_NW, 16), jnp.float32),
        mesh=plsc.VectorSubcoreMesh(core_axis_name="c", subcore_axis_name="s",
                                    num_cores=2, num_subcores=16),
        compiler_params=pltpu.CompilerParams(needs_layout_passes=False),
        scratch_types=[
            pltpu.VMEM((_N,), jnp.float32),
            pltpu.VMEM((_N,), jnp.float32),
            pltpu.VMEM((_N + 16,), jnp.int32),
            pltpu.VMEM((4096,), jnp.int32),
            pltpu.VMEM((16,), jnp.float32),
            pltpu.SemaphoreType.DMA,
            pltpu.SemaphoreType.DMA,
        ],
    )


def kernel(x):
    out = _sc_call()(x)
    return out[:, :_ROWS_PER_W].reshape(_ROWS)
